# baseline (speedup 1.0000x reference)
import functools

import jax
import jax.numpy as jnp
from jax import lax
from jax.experimental import pallas as pl
from jax.experimental.pallas import tpu as pltpu
from jax.experimental.pallas import tpu_sc as plsc

_B = 16384
_S = 50
_D = 32
_N = _B * _S
_NW = 32
_B_PER_W = _N // _NW
_CHUNK = 1024
_NC = _B_PER_W // _CHUNK
_NBUF = 3

_mesh = plsc.VectorSubcoreMesh(core_axis_name="c", subcore_axis_name="s")


@functools.partial(
    pl.kernel,
    mesh=_mesh,
    out_type=jax.ShapeDtypeStruct((_N, _D), jnp.float32),
    scratch_types=[
        pltpu.VMEM((_B_PER_W,), jnp.int32),
        pltpu.VMEM((_NBUF, _CHUNK, _D), jnp.float32),
        [pltpu.SemaphoreType.DMA] * _NBUF,
        [pltpu.SemaphoreType.DMA] * _NBUF,
    ],
    compiler_params=pltpu.CompilerParams(use_tc_tiling_on_sc=False),
)
def _sc_gather(table_hbm, idx_hbm, out_hbm, idx_all, rows, gsems, ssems):
    wid = lax.axis_index("s") * 2 + lax.axis_index("c")
    base = wid * _B_PER_W

    pltpu.sync_copy(idx_hbm.at[pl.ds(base, _B_PER_W)], idx_all)

    gathers = {}
    stores = {}

    def start_gather(c):
        gathers[c] = pltpu.async_copy(
            table_hbm.at[idx_all.at[pl.ds(c * _CHUNK, _CHUNK)]],
            rows.at[c % _NBUF],
            gsems[c % _NBUF])

    for c in range(_NBUF):
        start_gather(c)

    for c in range(_NC):
        if c > 0:
            stores[c - 1].wait()
            if c + _NBUF - 1 < _NC:
                start_gather(c + _NBUF - 1)
        gathers[c].wait()
        stores[c] = pltpu.async_copy(
            rows.at[c % _NBUF],
            out_hbm.at[pl.ds(base + c * _CHUNK, _CHUNK)],
            ssems[c % _NBUF])

    stores[_NC - 1].wait()


def _tc_transpose_body(rows_ref, out_ref):
    blk = rows_ref[...].reshape(_D, 4, _D)
    out_ref[...] = jnp.transpose(blk, (2, 0, 1)).reshape(_D, 128)


@functools.partial(jax.jit, donate_argnums=())
def _tc_transpose(lin128):
    ng_s, ng_b = _S, _B // 128
    return pl.pallas_call(
        _tc_transpose_body,
        grid=(ng_s, ng_b),
        in_specs=[pl.BlockSpec((_D, 128), lambda s, bb: (s * ng_b + bb, 0))],
        out_specs=pl.BlockSpec((_D, 128), lambda s, bb: (s, bb)),
        out_shape=jax.ShapeDtypeStruct((_S * _D, _B), jnp.float32),
    )(lin128)


def kernel(indices, embedding):
    idx = indices.T.reshape(-1).astype(jnp.int32)
    lin = _sc_gather(embedding, idx)
    lin128 = lin.reshape(_N * _D // 128, 128)
    o = _tc_transpose(lin128)
    return o.reshape(_S, _D, _B).transpose(2, 0, 1)


# speedup vs baseline: 6.2342x; 6.2342x over previous
import functools

import jax
import jax.numpy as jnp
from jax import lax
from jax.experimental import pallas as pl
from jax.experimental.pallas import tpu as pltpu
from jax.experimental.pallas import tpu_sc as plsc

_B = 16384
_S = 50
_D = 32
_N = _B * _S
_NW = 32
_B_PER_W = _N // _NW
_CHUNK = 1024
_NC = _B_PER_W // _CHUNK
_NBUF = 3

_mesh = plsc.VectorSubcoreMesh(core_axis_name="c", subcore_axis_name="s")


@functools.partial(
    pl.kernel,
    mesh=_mesh,
    out_type=jax.ShapeDtypeStruct((_N, 128), jnp.float32),
    scratch_types=[
        pltpu.VMEM((_B_PER_W,), jnp.int32),
        pltpu.VMEM((_NBUF, _CHUNK, _D), jnp.float32),
        [pltpu.SemaphoreType.DMA] * _NBUF,
        [pltpu.SemaphoreType.DMA] * _NBUF,
    ],
    compiler_params=pltpu.CompilerParams(use_tc_tiling_on_sc=False),
)
def _sc_gather(table_hbm, idx_hbm, out_hbm, idx_all, rows, gsems, ssems):
    wid = lax.axis_index("s") * 2 + lax.axis_index("c")
    base = wid * _B_PER_W

    pltpu.sync_copy(idx_hbm.at[pl.ds(base, _B_PER_W)], idx_all)

    gathers = {}
    stores = {}

    def start_gather(c):
        gathers[c] = pltpu.async_copy(
            table_hbm.at[idx_all.at[pl.ds(c * _CHUNK, _CHUNK)]],
            rows.at[c % _NBUF],
            gsems[c % _NBUF])

    for c in range(_NBUF):
        start_gather(c)

    for c in range(_NC):
        if c > 0:
            stores[c - 1].wait()
            if c + _NBUF - 1 < _NC:
                start_gather(c + _NBUF - 1)
        gathers[c].wait()
        stores[c] = pltpu.async_copy(
            rows.at[c % _NBUF],
            out_hbm.at[pl.ds(base + c * _CHUNK, _CHUNK), pl.ds(0, _D)],
            ssems[c % _NBUF])

    stores[_NC - 1].wait()


def _tc_transpose_body(rows_ref, out_ref):
    out_ref[...] = rows_ref[:, : _D].T


def _tc_transpose(lin_pad):
    return pl.pallas_call(
        _tc_transpose_body,
        grid=(_S,),
        in_specs=[pl.BlockSpec((_B, 128), lambda s: (s, 0))],
        out_specs=pl.BlockSpec((_D, _B), lambda s: (s, 0)),
        out_shape=jax.ShapeDtypeStruct((_S * _D, _B), jnp.float32),
    )(lin_pad)


def kernel(indices, embedding):
    idx = indices.T.reshape(-1).astype(jnp.int32)
    lin_pad = _sc_gather(embedding, idx)
    o = _tc_transpose(lin_pad)
    return o.reshape(_S, _D, _B).transpose(2, 0, 1)


# speedup vs baseline: 6.7416x; 1.0814x over previous
import functools

import jax
import jax.numpy as jnp
from jax import lax
from jax.experimental import pallas as pl
from jax.experimental.pallas import tpu as pltpu
from jax.experimental.pallas import tpu_sc as plsc

_B = 16384
_S = 50
_D = 32
_N = _B * _S
_NW = 32
_B_PER_W = _N // _NW
_CHUNK = 256
_NC = _B_PER_W // _CHUNK
_NBUF = 3
_V = 1000000

_mesh = plsc.VectorSubcoreMesh(core_axis_name="c", subcore_axis_name="s")


@functools.partial(
    pl.kernel,
    mesh=_mesh,
    out_type=jax.ShapeDtypeStruct((_N, 128), jnp.float32),
    scratch_types=[
        pltpu.VMEM((_B_PER_W,), jnp.int32),
        pltpu.VMEM((_NBUF, _CHUNK, 128), jnp.float32),
        [pltpu.SemaphoreType.DMA] * _NBUF,
        [pltpu.SemaphoreType.DMA] * _NBUF,
    ],
    compiler_params=pltpu.CompilerParams(use_tc_tiling_on_sc=False),
)
def _sc_gather(table_hbm, idx_hbm, out_hbm, idx_all, rows, gsems, ssems):
    wid = lax.axis_index("s") * 2 + lax.axis_index("c")
    base = wid * _B_PER_W

    pltpu.sync_copy(idx_hbm.at[pl.ds(base, _B_PER_W)], idx_all)

    gathers = {}
    stores = {}

    def start_gather(c):
        gathers[c] = pltpu.async_copy(
            table_hbm.at[idx_all.at[pl.ds(c * _CHUNK, _CHUNK)]],
            rows.at[c % _NBUF],
            gsems[c % _NBUF])

    for c in range(_NBUF):
        start_gather(c)

    for c in range(_NC):
        if c > 0:
            stores[c - 1].wait()
            if c + _NBUF - 1 < _NC:
                start_gather(c + _NBUF - 1)
        gathers[c].wait()
        stores[c] = pltpu.async_copy(
            rows.at[c % _NBUF],
            out_hbm.at[pl.ds(base + c * _CHUNK, _CHUNK)],
            ssems[c % _NBUF])

    stores[_NC - 1].wait()


def _tc_detile_body(col_ref, out_ref):
    out_ref[:, : _D] = col_ref[...].T


def _tc_detile(table_t):
    blk = 16384
    nb = -(-_V // blk)
    return pl.pallas_call(
        _tc_detile_body,
        grid=(nb,),
        in_specs=[pl.BlockSpec((_D, blk), lambda p: (0, p))],
        out_specs=pl.BlockSpec((blk, 128), lambda p: (p, 0)),
        out_shape=jax.ShapeDtypeStruct((_V, 128), jnp.float32),
    )(table_t)


def _tc_transpose_body(rows_ref, out_ref):
    out_ref[...] = rows_ref[:, : _D].T


def _tc_transpose(lin_pad):
    return pl.pallas_call(
        _tc_transpose_body,
        grid=(_S,),
        in_specs=[pl.BlockSpec((_B, 128), lambda s: (s, 0))],
        out_specs=pl.BlockSpec((_D, _B), lambda s: (s, 0)),
        out_shape=jax.ShapeDtypeStruct((_S * _D, _B), jnp.float32),
    )(lin_pad)


def kernel(indices, embedding):
    idx = indices.T.reshape(-1).astype(jnp.int32)
    table_pad = _tc_detile(embedding.T)
    lin_pad = _sc_gather(table_pad, idx)
    o = _tc_transpose(lin_pad)
    return o.reshape(_S, _D, _B).transpose(2, 0, 1)


# speedup vs baseline: 8.0371x; 1.1922x over previous
import functools

import jax
import jax.numpy as jnp
from jax import lax
from jax.experimental import pallas as pl
from jax.experimental.pallas import tpu as pltpu
from jax.experimental.pallas import tpu_sc as plsc

_B = 16384
_S = 50
_D = 32
_N = _B * _S
_NW = 32
_B_PER_W = _N // _NW
_CHUNK = 256
_NC = _B_PER_W // _CHUNK
_NBUF = 3
_V = 1000000

_mesh = plsc.VectorSubcoreMesh(core_axis_name="c", subcore_axis_name="s")


@functools.partial(
    pl.kernel,
    mesh=_mesh,
    out_type=jax.ShapeDtypeStruct((_N, 128), jnp.float32),
    scratch_types=[
        pltpu.VMEM((_B_PER_W,), jnp.int32),
        pltpu.VMEM((_NBUF, _CHUNK, 128), jnp.float32),
        [pltpu.SemaphoreType.DMA] * _NBUF,
        [pltpu.SemaphoreType.DMA] * _NBUF,
    ],
    compiler_params=pltpu.CompilerParams(use_tc_tiling_on_sc=False),
)
def _sc_gather(table_hbm, idx_hbm, out_hbm, idx_all, rows, gsems, ssems):
    wid = lax.axis_index("s") * 2 + lax.axis_index("c")
    base = wid * _B_PER_W

    pltpu.sync_copy(idx_hbm.at[pl.ds(base, _B_PER_W)], idx_all)

    gathers = {}
    stores = {}

    def start_gather(c):
        gathers[c] = pltpu.async_copy(
            table_hbm.at[idx_all.at[pl.ds(c * _CHUNK, _CHUNK)]],
            rows.at[c % _NBUF],
            gsems[c % _NBUF])

    for c in range(_NBUF):
        start_gather(c)

    for c in range(_NC):
        if c > 0:
            stores[c - 1].wait()
            if c + _NBUF - 1 < _NC:
                start_gather(c + _NBUF - 1)
        gathers[c].wait()
        stores[c] = pltpu.async_copy(
            rows.at[c % _NBUF, :, pl.ds(0, _D)],
            out_hbm.at[pl.ds(base + c * _CHUNK, _CHUNK), pl.ds(0, _D)],
            ssems[c % _NBUF])

    stores[_NC - 1].wait()


def _tc_detile_body(col_ref, out_ref):
    out_ref[:, : _D] = col_ref[...].T


def _tc_detile(table_t):
    blk = 16384
    nb = -(-_V // blk)
    return pl.pallas_call(
        _tc_detile_body,
        grid=(nb,),
        in_specs=[pl.BlockSpec((_D, blk), lambda p: (0, p))],
        out_specs=pl.BlockSpec((blk, 128), lambda p: (p, 0)),
        out_shape=jax.ShapeDtypeStruct((_V, 128), jnp.float32),
    )(table_t)


def _tc_transpose_body(rows_ref, out_ref):
    out_ref[...] = rows_ref[:, : _D].T


def _tc_transpose(lin_pad):
    return pl.pallas_call(
        _tc_transpose_body,
        grid=(_S,),
        in_specs=[pl.BlockSpec((_B, 128), lambda s: (s, 0))],
        out_specs=pl.BlockSpec((_D, _B), lambda s: (s, 0)),
        out_shape=jax.ShapeDtypeStruct((_S * _D, _B), jnp.float32),
    )(lin_pad)


def kernel(indices, embedding):
    idx = indices.T.reshape(-1).astype(jnp.int32)
    table_pad = _tc_detile(embedding.T)
    lin_pad = _sc_gather(table_pad, idx)
    o = _tc_transpose(lin_pad)
    return o.reshape(_S, _D, _B).transpose(2, 0, 1)
